# baseline (speedup 1.0000x reference)
import jax
import jax.numpy as jnp
from jax import lax
from jax.experimental import pallas as pl
from jax.experimental.pallas import tpu as pltpu
from jax.experimental.pallas import tpu_sc as plsc

_NC = 2
_NS = 16
_NW = _NC * _NS
_LANES = 16

_EMBED = 128
_BATCH = 16384
_B_PER_W = _BATCH // _NW
_CHUNK = 128
_NCHUNKS = _B_PER_W // _CHUNK
_NBUF = 2


def _sc_body(idx_hbm, utab_hbm, itab_hbm, out_hbm,
             idx_v, u0, u1, i0, i1, acc_v, out_v,
             sem_idx, sem_out, su0, su1, si0, si1):
    u_bufs = (u0, u1)
    i_bufs = (i0, i1)
    u_sems = (su0, su1)
    i_sems = (si0, si1)
    wid = lax.axis_index("s") * _NC + lax.axis_index("c")

    pltpu.async_copy(idx_hbm.at[wid], idx_v, sem_idx).wait()

    lanes = lax.iota(jnp.int32, _LANES)
    zeros = jnp.zeros((_LANES,), jnp.int32)

    pieces = [(0, 32), (32, 32), (64, 64), (128, 128), (256, 128), (384, 128)]
    uflat = idx_v.at[0]
    iflat = idx_v.at[1]

    def issue(p):
        s = p % _NBUF
        lo, n = pieces[p]
        return (pltpu.async_copy(utab_hbm.at[uflat.at[pl.ds(lo, n)]],
                                 u_bufs[s].at[pl.ds(0, n)], u_sems[s]),
                pltpu.async_copy(itab_hbm.at[iflat.at[pl.ds(lo, n)]],
                                 i_bufs[s].at[pl.ds(0, n)], i_sems[s]))

    out_cps = []
    pending = [issue(p) for p in range(_NBUF)]
    for c in range(len(pieces)):
        cp_u, cp_i = pending[c]
        cp_u.wait()
        cp_i.wait()
        u_v = u_bufs[c % _NBUF]
        i_v = i_bufs[c % _NBUF]

        lo, n = pieces[c]

        def group_body(g, _, lo=lo, u_v=u_v, i_v=i_v):
            base = g * _LANES
            for rr in range(_LANES):
                r = base + rr
                acc = u_v[r, pl.ds(0, _LANES)] * i_v[r, pl.ds(0, _LANES)]
                for j in range(1, _EMBED // _LANES):
                    acc = acc + (u_v[r, pl.ds(j * _LANES, _LANES)]
                                 * i_v[r, pl.ds(j * _LANES, _LANES)])
                acc_v[rr] = acc
            res = plsc.load_gather(acc_v, [lanes, zeros])
            for j in range(1, _LANES):
                res = res + plsc.load_gather(
                    acc_v, [lanes, jnp.full((_LANES,), j, jnp.int32)])
            out_v[pl.ds(lo + base, _LANES)] = res
            return 0

        lax.fori_loop(0, n // _LANES, group_body, 0)
        out_cps.append(
            pltpu.async_copy(out_v.at[pl.ds(lo, n)],
                             out_hbm.at[wid, lo // _CHUNK].at[pl.ds(lo % _CHUNK, n)],
                             sem_out))
        if c + _NBUF < len(pieces):
            pending.append(issue(c + _NBUF))

    for cp in out_cps:
        cp.wait()


@jax.jit
def _cf_dot(idx, user_table, item_table):
    mesh = plsc.VectorSubcoreMesh(core_axis_name="c", subcore_axis_name="s",
                                  num_cores=_NC, num_subcores=_NS)
    k = pl.kernel(
        _sc_body,
        out_type=jax.ShapeDtypeStruct((_NW, _NCHUNKS, _CHUNK), jnp.float32),
        mesh=mesh,
        scratch_types=[
            pltpu.VMEM((2, _B_PER_W), jnp.int32),
            *[pltpu.VMEM((_CHUNK, _EMBED), jnp.float32) for _ in range(2 * _NBUF)],
            pltpu.VMEM((_LANES, _LANES), jnp.float32),
            pltpu.VMEM((_B_PER_W,), jnp.float32),
            *[pltpu.SemaphoreType.DMA for _ in range(2 + 2 * _NBUF)],
        ],
        compiler_params=pltpu.CompilerParams(needs_layout_passes=False),
    )
    return k(idx, user_table, item_table)


def kernel(inputs, user_table, item_table):
    idx = inputs.astype(jnp.int32)
    uidx = idx[:, 0].reshape(_NW, 1, _B_PER_W)
    iidx = idx[:, 1].reshape(_NW, 1, _B_PER_W)
    both = jnp.concatenate([uidx, iidx], axis=1)
    out = _cf_dot(both, user_table, item_table)
    return out.reshape(_BATCH)


# speedup vs baseline: 1.0055x; 1.0055x over previous
import jax
import jax.numpy as jnp
from jax import lax
from jax.experimental import pallas as pl
from jax.experimental.pallas import tpu as pltpu
from jax.experimental.pallas import tpu_sc as plsc

_NC = 2
_NS = 16
_NW = _NC * _NS
_LANES = 16

_EMBED = 128
_BATCH = 16384
_B_PER_W = _BATCH // _NW
_CHUNK = 128
_NCHUNKS = _B_PER_W // _CHUNK
_NBUF = 2


def _sc_body(idx_hbm, utab_hbm, itab_hbm, out_hbm,
             idx_v, u0, u1, i0, i1, acc_v, out_v,
             sem_idx, sem_out, su0, su1, si0, si1):
    u_bufs = (u0, u1)
    i_bufs = (i0, i1)
    u_sems = (su0, su1)
    i_sems = (si0, si1)
    wid = lax.axis_index("s") * _NC + lax.axis_index("c")

    pltpu.async_copy(idx_hbm.at[wid], idx_v, sem_idx).wait()

    lanes = lax.iota(jnp.int32, _LANES)
    zeros = jnp.zeros((_LANES,), jnp.int32)

    pieces = [(0, 128), (128, 128), (256, 128), (384, 128)]
    uflat = idx_v.at[0]
    iflat = idx_v.at[1]

    def issue(p):
        s = p % _NBUF
        lo, n = pieces[p]
        return (pltpu.async_copy(utab_hbm.at[uflat.at[pl.ds(lo, n)]],
                                 u_bufs[s].at[pl.ds(0, n)], u_sems[s]),
                pltpu.async_copy(itab_hbm.at[iflat.at[pl.ds(lo, n)]],
                                 i_bufs[s].at[pl.ds(0, n)], i_sems[s]))

    out_cps = []
    pending = [issue(p) for p in range(_NBUF)]
    for c in range(len(pieces)):
        cp_u, cp_i = pending[c]
        cp_u.wait()
        cp_i.wait()
        u_v = u_bufs[c % _NBUF]
        i_v = i_bufs[c % _NBUF]

        lo, n = pieces[c]

        def group_body(g, _, lo=lo, u_v=u_v, i_v=i_v):
            base = g * _LANES
            for rr in range(_LANES):
                r = base + rr
                acc = u_v[r, pl.ds(0, _LANES)] * i_v[r, pl.ds(0, _LANES)]
                for j in range(1, _EMBED // _LANES):
                    acc = acc + (u_v[r, pl.ds(j * _LANES, _LANES)]
                                 * i_v[r, pl.ds(j * _LANES, _LANES)])
                acc_v[rr] = acc
            res = plsc.load_gather(acc_v, [lanes, zeros])
            for j in range(1, _LANES):
                res = res + plsc.load_gather(
                    acc_v, [lanes, jnp.full((_LANES,), j, jnp.int32)])
            out_v[pl.ds(lo + base, _LANES)] = res
            return 0

        lax.fori_loop(0, n // _LANES, group_body, 0)
        out_cps.append(
            pltpu.async_copy(out_v.at[pl.ds(lo, n)],
                             out_hbm.at[wid, lo // _CHUNK].at[pl.ds(lo % _CHUNK, n)],
                             sem_out))
        if c + _NBUF < len(pieces):
            pending.append(issue(c + _NBUF))

    for cp in out_cps:
        cp.wait()


@jax.jit
def _cf_dot(idx, user_table, item_table):
    mesh = plsc.VectorSubcoreMesh(core_axis_name="c", subcore_axis_name="s",
                                  num_cores=_NC, num_subcores=_NS)
    k = pl.kernel(
        _sc_body,
        out_type=jax.ShapeDtypeStruct((_NW, _NCHUNKS, _CHUNK), jnp.float32),
        mesh=mesh,
        scratch_types=[
            pltpu.VMEM((2, _B_PER_W), jnp.int32),
            *[pltpu.VMEM((_CHUNK, _EMBED), jnp.float32) for _ in range(2 * _NBUF)],
            pltpu.VMEM((_LANES, _LANES), jnp.float32),
            pltpu.VMEM((_B_PER_W,), jnp.float32),
            *[pltpu.SemaphoreType.DMA for _ in range(2 + 2 * _NBUF)],
        ],
        compiler_params=pltpu.CompilerParams(
            needs_layout_passes=False,
            disable_bounds_checks=True,
            disable_semaphore_checks=True,
        ),
    )
    return k(idx, user_table, item_table)


def kernel(inputs, user_table, item_table):
    idx = inputs.astype(jnp.int32)
    uidx = idx[:, 0].reshape(_NW, 1, _B_PER_W)
    iidx = idx[:, 1].reshape(_NW, 1, _B_PER_W)
    both = jnp.concatenate([uidx, iidx], axis=1)
    out = _cf_dot(both, user_table, item_table)
    return out.reshape(_BATCH)
